# baseline (speedup 1.0000x reference)
import functools

import jax
import jax.numpy as jnp
from jax import lax
from jax.experimental import pallas as pl
from jax.experimental.pallas import tpu as pltpu
from jax.experimental.pallas import tpu_sc as plsc

N_WORKERS = 32
LANES = 16
D_MODEL = 64
SCALE = 8.0

CHUNK = 400
GSUB = 80
NSUB = CHUNK // GSUB


@functools.partial(jax.jit, static_argnums=(3,))
def _embed(token_table, idx2d, pos, t_cur):
    n_idx = idx2d.shape[0]
    per_worker = n_idx // N_WORKERS
    n_chunks = per_worker // CHUNK
    reps = CHUNK // t_cur
    mesh = plsc.VectorSubcoreMesh(core_axis_name="c", subcore_axis_name="s")

    @functools.partial(
        pl.kernel,
        out_type=jax.ShapeDtypeStruct((n_idx, D_MODEL), jnp.float32),
        mesh=mesh,
        scratch_types=[
            pltpu.VMEM((CHUNK,), jnp.int32),
            pltpu.VMEM((CHUNK, D_MODEL), jnp.float32),
            pltpu.VMEM((t_cur, D_MODEL), jnp.float32),
            pltpu.SemaphoreType.DMA,
        ],
        compiler_params=pltpu.CompilerParams(use_tc_tiling_on_sc=False),
    )
    def k(tok_hbm, idx_hbm, pos_hbm, out_hbm, idx_v, rows_v, pos_v, sem):
        wid = lax.axis_index("s") * 2 + lax.axis_index("c")
        base = wid * per_worker
        pltpu.sync_copy(pos_hbm, pos_v)

        @pl.loop(0, n_chunks)
        def _(ci):
            start = base + ci * CHUNK
            pltpu.sync_copy(idx_hbm.at[pl.ds(start, CHUNK)], idx_v)
            copies = [
                pltpu.async_copy(
                    tok_hbm.at[idx_v.at[pl.ds(g * GSUB, GSUB)]],
                    rows_v.at[pl.ds(g * GSUB, GSUB)],
                    sem,
                )
                for g in range(NSUB)
            ]
            for cp in copies:
                cp.wait()

            for rep in range(reps):
                @pl.loop(0, t_cur)
                def _(t):
                    row = rows_v.at[rep * t_cur + t]
                    prow = pos_v.at[t]
                    for c in range(D_MODEL // LANES):
                        sl = pl.ds(c * LANES, LANES)
                        row[sl] = row[sl] * SCALE + prow[sl]

            pltpu.sync_copy(rows_v, out_hbm.at[pl.ds(start, CHUNK)])

    return k(token_table, idx2d, pos)


def kernel(x, token_table, pos_table):
    b, t_cur = x.shape
    n_idx = b * t_cur
    idx2d = x.astype(jnp.int32).reshape(n_idx)
    pos = pos_table[:t_cur]
    out = _embed(token_table, idx2d, pos, t_cur)
    return out.reshape(b, t_cur, D_MODEL)


# speedup vs baseline: 1.1713x; 1.1713x over previous
import functools

import jax
import jax.numpy as jnp
from jax import lax
from jax.experimental import pallas as pl
from jax.experimental.pallas import tpu as pltpu
from jax.experimental.pallas import tpu_sc as plsc

N_WORKERS = 32
LANES = 16
D_MODEL = 64
SCALE = 8.0

CHUNK = 400
GSUB = 80
NSUB = CHUNK // GSUB
NIBUF = 4


@functools.partial(jax.jit, static_argnums=(3,))
def _embed(token_table, idx_flat, pos, t_cur):
    n_idx = idx_flat.shape[0]
    per_worker = n_idx // N_WORKERS
    n_chunks = per_worker // CHUNK
    assert CHUNK // t_cur == 2 and n_chunks % NIBUF == 0
    mesh = plsc.VectorSubcoreMesh(core_axis_name="c", subcore_axis_name="s")

    @functools.partial(
        pl.kernel,
        out_type=jax.ShapeDtypeStruct((n_idx, D_MODEL), jnp.float32),
        mesh=mesh,
        scratch_types=(
            [pltpu.VMEM((CHUNK,), jnp.int32)] * NIBUF
            + [pltpu.VMEM((CHUNK, D_MODEL), jnp.float32)] * 4
            + [pltpu.VMEM((t_cur, D_MODEL), jnp.float32)]
            + [pltpu.SemaphoreType.DMA] * (NIBUF + 4)
        ),
        compiler_params=pltpu.CompilerParams(use_tc_tiling_on_sc=False),
    )
    def k(tok_hbm, idx_hbm, pos_hbm, out_hbm,
          ix0, ix1, ix2, ix3, rv0, rv1, ov0, ov1, pos_v,
          is0, is1, is2, is3, gs0, gs1, ws0, ws1):
        ixs = (ix0, ix1, ix2, ix3)
        isems = (is0, is1, is2, is3)
        rvs = (rv0, rv1)
        ovs = (ov0, ov1)
        gsems = (gs0, gs1)
        wsems = (ws0, ws1)

        wid = lax.axis_index("s") * 2 + lax.axis_index("c")
        base = wid * per_worker
        pltpu.sync_copy(pos_hbm, pos_v)

        def idx_start(ci, slot):
            pltpu.async_copy(
                idx_hbm.at[pl.ds(base + ci * CHUNK, CHUNK)], ixs[slot],
                isems[slot])

        def idx_wait(slot):
            pltpu.make_async_copy(
                idx_hbm.at[pl.ds(base, CHUNK)], ixs[slot], isems[slot]).wait()

        def gathers_start(slot, b):
            for g in range(NSUB):
                pltpu.async_copy(
                    tok_hbm.at[ixs[slot].at[pl.ds(g * GSUB, GSUB)]],
                    rvs[b].at[pl.ds(g * GSUB, GSUB)],
                    gsems[b])

        def gathers_wait(slot, b):
            for g in range(NSUB):
                pltpu.make_async_copy(
                    tok_hbm.at[ixs[slot].at[pl.ds(g * GSUB, GSUB)]],
                    rvs[b].at[pl.ds(g * GSUB, GSUB)],
                    gsems[b]).wait()

        def wb_start(ci, b):
            pltpu.async_copy(
                ovs[b], out_hbm.at[pl.ds(base + ci * CHUNK, CHUNK)], wsems[b])

        def wb_wait(b):
            pltpu.make_async_copy(
                ovs[b], out_hbm.at[pl.ds(base, CHUNK)], wsems[b]).wait()

        def compute(b):
            rv, ov = rvs[b], ovs[b]

            @plsc.parallel_loop(0, t_cur, unroll=2)
            def _(t):
                prow = pos_v.at[t]
                ra, rb = rv.at[t], rv.at[t_cur + t]
                oa, ob = ov.at[t], ov.at[t_cur + t]
                for c in range(D_MODEL // LANES):
                    sl = pl.ds(c * LANES, LANES)
                    p = prow[sl]
                    oa[sl] = ra[sl] * SCALE + p
                    ob[sl] = rb[sl] * SCALE + p

        for s in range(NIBUF):
            idx_start(s, s)
        idx_wait(0)
        gathers_start(0, 0)
        idx_wait(1)
        gathers_start(1, 1)

        @pl.loop(0, n_chunks, step=NIBUF)
        def _(ci0):
            for j in range(NIBUF):
                ci = ci0 + j
                b = j % 2
                gathers_wait(j, b)
                idx_start(jnp.minimum(ci + NIBUF, n_chunks - 1), j)
                if j < 2:
                    @pl.when(ci >= 2)
                    def _():
                        wb_wait(b)
                else:
                    wb_wait(b)
                compute(b)
                wb_start(ci, b)
                idx_wait((j + 2) % NIBUF)
                gathers_start((j + 2) % NIBUF, b)

        idx_wait(2)
        idx_wait(3)
        gathers_wait(0, 0)
        gathers_wait(1, 1)
        wb_wait(0)
        wb_wait(1)

    return k(token_table, idx_flat, pos)


def kernel(x, token_table, pos_table):
    b, t_cur = x.shape
    idx_flat = x.astype(jnp.int32).reshape(b * t_cur)
    pos = pos_table[:t_cur]
    out = _embed(token_table, idx_flat, pos, t_cur)
    return out.reshape(b, t_cur, D_MODEL)
